# baseline (speedup 1.0000x reference)
import functools

import jax
import jax.numpy as jnp
from jax import lax
from jax.experimental import pallas as pl
from jax.experimental.pallas import tpu as pltpu
from jax.experimental.pallas import tpu_sc as plsc

_NC = 2
_NS = 16
_NW = _NC * _NS
_LANES = 16
_CHUNK = 16384
_NBUF = 4


def _sc_clip(total_elems):
    per_worker = total_elems // _NW
    n_chunks = per_worker // _CHUNK
    assert per_worker % _CHUNK == 0 and n_chunks % _NBUF == 0

    mesh = plsc.VectorSubcoreMesh(core_axis_name="c", subcore_axis_name="s")

    @functools.partial(
        pl.kernel,
        out_type=jax.ShapeDtypeStruct((total_elems,), jnp.float32),
        mesh=mesh,
        scratch_types=[
            [pltpu.VMEM((_CHUNK,), jnp.float32)] * _NBUF,
            [pltpu.SemaphoreType.DMA] * _NBUF,
            [pltpu.SemaphoreType.DMA] * _NBUF,
        ],
    )
    def body(x_hbm, o_hbm, bufs, in_sems, out_sems):
        wid = lax.axis_index("s") * _NC + lax.axis_index("c")
        base = wid * per_worker

        def compute(buf):
            @pl.loop(0, _CHUNK, step=_LANES, unroll=8)
            def _(j):
                v = buf[pl.ds(j, _LANES)]
                buf[pl.ds(j, _LANES)] = jnp.minimum(jnp.maximum(v, 0.0), 1.0)

        def chunk_slice(idx):
            return pl.ds(base + idx * _CHUNK, _CHUNK)

        pltpu.async_copy(x_hbm.at[chunk_slice(0)], bufs[0], in_sems[0])
        pltpu.async_copy(x_hbm.at[chunk_slice(1)], bufs[1], in_sems[1])

        @pl.loop(0, n_chunks, step=_NBUF)
        def _(i):
            for b in range(_NBUF):
                idx = i + b
                pf = (b + 2) % _NBUF

                @pl.when(idx + 2 < n_chunks)
                def _():
                    @pl.when(idx >= 2)
                    def _():
                        pltpu.make_async_copy(
                            bufs[pf], o_hbm.at[chunk_slice(idx - 2)],
                            out_sems[pf]).wait()
                    pltpu.async_copy(
                        x_hbm.at[chunk_slice(idx + 2)], bufs[pf], in_sems[pf])

                pltpu.make_async_copy(
                    x_hbm.at[chunk_slice(idx)], bufs[b], in_sems[b]).wait()
                compute(bufs[b])
                pltpu.async_copy(
                    bufs[b], o_hbm.at[chunk_slice(idx)], out_sems[b])

        for b in range(_NBUF):
            pltpu.make_async_copy(
                bufs[b], o_hbm.at[chunk_slice(b)], out_sems[b]).wait()

    return body


def kernel(x):
    n_rows, n_cols = x.shape
    total = n_rows * n_cols
    y = _sc_clip(total)(x.reshape(total))
    return y.reshape(n_rows, n_cols)


# speedup vs baseline: 1.0000x; 1.0000x over previous
import functools

import jax
import jax.numpy as jnp
from jax import lax
from jax.experimental import pallas as pl
from jax.experimental.pallas import tpu as pltpu
from jax.experimental.pallas import tpu_sc as plsc

_NC = 2
_NS = 16
_NW = _NC * _NS
_LANES = 16
_CHUNK = 16384
_NBUF = 4


def _sc_clip(total_elems):
    per_worker = total_elems // _NW
    n_chunks = per_worker // _CHUNK
    assert per_worker % _CHUNK == 0 and n_chunks % _NBUF == 0

    mesh = plsc.VectorSubcoreMesh(core_axis_name="c", subcore_axis_name="s")

    @functools.partial(
        pl.kernel,
        out_type=jax.ShapeDtypeStruct((total_elems,), jnp.float32),
        mesh=mesh,
        scratch_types=[
            [pltpu.VMEM((_CHUNK,), jnp.float32)] * _NBUF,
            [pltpu.SemaphoreType.DMA] * _NBUF,
            [pltpu.SemaphoreType.DMA] * _NBUF,
        ],
    )
    def body(x_hbm, o_hbm, bufs, in_sems, out_sems):
        wid = lax.axis_index("s") * _NC + lax.axis_index("c")
        base = wid * per_worker

        def compute(buf):
            @pl.loop(0, _CHUNK, step=_LANES, unroll=8)
            def _(j):
                v = buf[pl.ds(j, _LANES)]
                buf[pl.ds(j, _LANES)] = jnp.minimum(jnp.maximum(v, 0.0), 1.0)

        def chunk_slice(idx):
            return pl.ds(base + idx * _CHUNK, _CHUNK)

        pltpu.async_copy(x_hbm.at[chunk_slice(0)], bufs[0], in_sems[0])
        pltpu.async_copy(x_hbm.at[chunk_slice(1)], bufs[1], in_sems[1])

        @pl.loop(0, n_chunks, step=_NBUF)
        def _(i):
            for b in range(_NBUF):
                idx = i + b
                pf = (b + 2) % _NBUF

                @pl.when(idx + 2 < n_chunks)
                def _():
                    @pl.when(idx >= 2)
                    def _():
                        pltpu.make_async_copy(
                            bufs[pf], o_hbm.at[chunk_slice(idx - 2)],
                            out_sems[pf]).wait()
                    pltpu.async_copy(
                        x_hbm.at[chunk_slice(idx + 2)], bufs[pf], in_sems[pf])

                pltpu.make_async_copy(
                    x_hbm.at[chunk_slice(idx)], bufs[b], in_sems[b]).wait()
                pltpu.async_copy(
                    bufs[b], o_hbm.at[chunk_slice(idx)], out_sems[b])

        for b in range(_NBUF):
            pltpu.make_async_copy(
                bufs[b], o_hbm.at[chunk_slice(b)], out_sems[b]).wait()

    return body


def kernel(x):
    n_rows, n_cols = x.shape
    total = n_rows * n_cols
    y = _sc_clip(total)(x.reshape(total))
    return y.reshape(n_rows, n_cols)


# speedup vs baseline: 1.0034x; 1.0034x over previous
import functools

import jax
import jax.numpy as jnp
from jax import lax
from jax.experimental import pallas as pl
from jax.experimental.pallas import tpu as pltpu
from jax.experimental.pallas import tpu_sc as plsc

_NC = 2
_NS = 16
_NW = _NC * _NS
_LANES = 16
_CHUNK = 32768


def _sc_clip(total_elems):
    per_worker = total_elems // _NW
    n_chunks = per_worker // _CHUNK
    assert per_worker % _CHUNK == 0

    mesh = plsc.VectorSubcoreMesh(core_axis_name="c", subcore_axis_name="s")

    @functools.partial(
        pl.kernel,
        out_type=jax.ShapeDtypeStruct((total_elems,), jnp.float32),
        mesh=mesh,
        scratch_types=[
            [pltpu.VMEM((_CHUNK,), jnp.float32)] * 2,
            [pltpu.SemaphoreType.DMA] * 2,
            [pltpu.SemaphoreType.DMA] * 2,
        ],
    )
    def body(x_hbm, o_hbm, bufs, in_sems, out_sems):
        wid = lax.axis_index("s") * _NC + lax.axis_index("c")
        base = wid * per_worker

        def compute(buf):
            @pl.loop(0, _CHUNK, step=_LANES, unroll=8)
            def _(j):
                v = buf[pl.ds(j, _LANES)]
                buf[pl.ds(j, _LANES)] = jnp.minimum(jnp.maximum(v, 0.0), 1.0)

        def chunk_slice(idx):
            return pl.ds(base + idx * _CHUNK, _CHUNK)

        pltpu.async_copy(x_hbm.at[chunk_slice(0)], bufs[0], in_sems[0])

        @pl.loop(0, n_chunks, step=2)
        def _(i):
            for b in range(2):
                idx = i + b
                nb = 1 - b

                @pl.when(idx + 1 < n_chunks)
                def _():
                    @pl.when(idx >= 1)
                    def _():
                        pltpu.make_async_copy(
                            bufs[nb], o_hbm.at[chunk_slice(idx - 1)],
                            out_sems[nb]).wait()
                    pltpu.async_copy(
                        x_hbm.at[chunk_slice(idx + 1)], bufs[nb], in_sems[nb])

                pltpu.make_async_copy(
                    x_hbm.at[chunk_slice(idx)], bufs[b], in_sems[b]).wait()
                pltpu.async_copy(
                    bufs[b], o_hbm.at[chunk_slice(idx)], out_sems[b])

        for b in range(2):
            pltpu.make_async_copy(
                bufs[b], o_hbm.at[chunk_slice(b)], out_sems[b]).wait()

    return body


def kernel(x):
    n_rows, n_cols = x.shape
    total = n_rows * n_cols
    y = _sc_clip(total)(x.reshape(total))
    return y.reshape(n_rows, n_cols)


# speedup vs baseline: 1.1173x; 1.1135x over previous
import functools

import jax
import jax.numpy as jnp
from jax import lax
from jax.experimental import pallas as pl
from jax.experimental.pallas import tpu as pltpu
from jax.experimental.pallas import tpu_sc as plsc

_NC = 2
_NS = 16
_NW = _NC * _NS
_LANES = 16
_CHUNK = 32768
_SC_ROWS = 6144
_TC_BLOCK_ROWS = 1024


def _sc_clip(total_elems):
    per_worker = total_elems // _NW
    n_chunks = per_worker // _CHUNK
    assert per_worker % _CHUNK == 0 and n_chunks % 2 == 0

    mesh = plsc.VectorSubcoreMesh(core_axis_name="c", subcore_axis_name="s")

    @functools.partial(
        pl.kernel,
        out_type=jax.ShapeDtypeStruct((total_elems,), jnp.float32),
        mesh=mesh,
        scratch_types=[
            [pltpu.VMEM((_CHUNK,), jnp.float32)] * 2,
            [pltpu.SemaphoreType.DMA] * 2,
            [pltpu.SemaphoreType.DMA] * 2,
        ],
    )
    def body(x_hbm, o_hbm, bufs, in_sems, out_sems):
        wid = lax.axis_index("s") * _NC + lax.axis_index("c")
        base = wid * per_worker

        def compute(buf):
            @pl.loop(0, _CHUNK, step=_LANES, unroll=8)
            def _(j):
                v = buf[pl.ds(j, _LANES)]
                buf[pl.ds(j, _LANES)] = jnp.minimum(jnp.maximum(v, 0.0), 1.0)

        def chunk_slice(idx):
            return pl.ds(base + idx * _CHUNK, _CHUNK)

        pltpu.async_copy(x_hbm.at[chunk_slice(0)], bufs[0], in_sems[0])

        @pl.loop(0, n_chunks, step=2)
        def _(i):
            for b in range(2):
                idx = i + b
                nb = 1 - b

                @pl.when(idx + 1 < n_chunks)
                def _():
                    @pl.when(idx >= 1)
                    def _():
                        pltpu.make_async_copy(
                            bufs[nb], o_hbm.at[chunk_slice(idx - 1)],
                            out_sems[nb]).wait()
                    pltpu.async_copy(
                        x_hbm.at[chunk_slice(idx + 1)], bufs[nb], in_sems[nb])

                pltpu.make_async_copy(
                    x_hbm.at[chunk_slice(idx)], bufs[b], in_sems[b]).wait()
                compute(bufs[b])
                pltpu.async_copy(
                    bufs[b], o_hbm.at[chunk_slice(idx)], out_sems[b])

        for b in range(2):
            pltpu.make_async_copy(
                bufs[b], o_hbm.at[chunk_slice(b)], out_sems[b]).wait()

    return body


def _tc_clip_kernel(x_ref, o_ref):
    o_ref[...] = jnp.clip(x_ref[...], 0.0, 1.0)


def _tc_clip(x, row_start, n_rows):
    n_cols = x.shape[1]
    grid = (n_rows // _TC_BLOCK_ROWS,)
    blk = row_start // _TC_BLOCK_ROWS
    return pl.pallas_call(
        _tc_clip_kernel,
        grid=grid,
        in_specs=[pl.BlockSpec((_TC_BLOCK_ROWS, n_cols),
                               lambda i, blk=blk: (i + blk, 0))],
        out_specs=pl.BlockSpec((_TC_BLOCK_ROWS, n_cols), lambda i: (i, 0)),
        out_shape=jax.ShapeDtypeStruct((n_rows, n_cols), x.dtype),
        compiler_params=pltpu.CompilerParams(
            dimension_semantics=("arbitrary",),
        ),
    )(x)


def kernel(x):
    n_rows, n_cols = x.shape
    sc_elems = _SC_ROWS * n_cols
    y_sc = _sc_clip(sc_elems)(x.reshape(n_rows * n_cols))
    y_tc = _tc_clip(x, _SC_ROWS, n_rows - _SC_ROWS)
    return jnp.concatenate(
        [y_sc.reshape(_SC_ROWS, n_cols), y_tc], axis=0)


# speedup vs baseline: 3.9743x; 3.5570x over previous
import jax
import jax.numpy as jnp
from jax.experimental import pallas as pl
from jax.experimental.pallas import tpu as pltpu

_CH_ROWS = 512
_NBUF = 6


def _body(x_hbm, o_hbm, buf, in_sems, out_sems):
    n_rows = x_hbm.shape[0]
    n_chunks = n_rows // _CH_ROWS

    def in_copy(idx, b):
        return pltpu.make_async_copy(
            x_hbm.at[pl.ds(idx * _CH_ROWS, _CH_ROWS), :],
            buf.at[b], in_sems.at[b])

    def out_copy(idx, b):
        return pltpu.make_async_copy(
            buf.at[b],
            o_hbm.at[pl.ds(idx * _CH_ROWS, _CH_ROWS), :],
            out_sems.at[b])

    in_copy(0, 0).start()
    in_copy(1, 1).start()

    def step(idx, _):
        b = jax.lax.rem(idx, _NBUF)
        pf = jax.lax.rem(idx + 2, _NBUF)

        @pl.when(idx + 2 < n_chunks)
        def _():
            @pl.when(idx + 2 >= _NBUF)
            def _():
                out_copy(idx + 2 - _NBUF, pf).wait()
            in_copy(idx + 2, pf).start()

        in_copy(idx, b).wait()
        buf[b] = jnp.clip(buf[b], 0.0, 1.0)
        out_copy(idx, b).start()
        return ()

    jax.lax.fori_loop(0, n_chunks, step, (), unroll=False)

    for i in range(_NBUF):
        idx = n_chunks - _NBUF + i
        out_copy(idx, idx % _NBUF).wait()


def kernel(x):
    n_rows, n_cols = x.shape
    return pl.pallas_call(
        _body,
        in_specs=[pl.BlockSpec(memory_space=pl.ANY)],
        out_specs=pl.BlockSpec(memory_space=pl.ANY),
        out_shape=jax.ShapeDtypeStruct((n_rows, n_cols), x.dtype),
        scratch_shapes=[
            pltpu.VMEM((_NBUF, _CH_ROWS, n_cols), x.dtype),
            pltpu.SemaphoreType.DMA((_NBUF,)),
            pltpu.SemaphoreType.DMA((_NBUF,)),
        ],
        compiler_params=pltpu.CompilerParams(
            vmem_limit_bytes=60 * 1024 * 1024,
        ),
    )(x)
